# baseline (speedup 1.0000x reference)
import functools

import jax
import jax.numpy as jnp
from jax import lax
from jax.experimental import pallas as pl
from jax.experimental.pallas import tpu as pltpu
from jax.experimental.pallas import tpu_sc as plsc

f32 = jnp.float32
i32 = jnp.int32

N = 10000
E = 320000
G = 64
F = 128
NP = 10240
NC = 2
NS = 16
NW = NC * NS
NB = NP // NW
CAPB = 12288
EB = NW * CAPB
EWS = EB // NW
CB = 2048
NR = NP // 128
JB = 8
DPAD = 16384

_MESH = plsc.VectorSubcoreMesh(
    core_axis_name="c", subcore_axis_name="s", num_cores=NC, num_subcores=NS)
_SC_PARAMS = pltpu.CompilerParams(needs_layout_passes=False)


def _zero16():
    return jnp.zeros((16,), f32)


def _zero_2d(ref, nrows):
    def body(it, _):
        r = it >> 3
        c = (it & 7) * 16
        ref[r, pl.ds(c, 16)] = _zero16()
        return 0
    lax.fori_loop(0, nrows * 8, body, 0)


def _fill_rowidx(ref, nrows):
    def body(it, _):
        ref[pl.ds(it * 16, 16)] = lax.iota(i32, 16) + it * 16
        return 0
    lax.fori_loop(0, nrows // 16, body, 0)


def _rc(v16):
    return lax.shift_right_logical(v16, 7), lax.bitwise_and(v16, 127)


@functools.partial(
    pl.kernel,
    out_type=[
        jax.ShapeDtypeStruct((EB,), i32),
        jax.ShapeDtypeStruct((EB,), i32),
        jax.ShapeDtypeStruct((EB,), f32),
        jax.ShapeDtypeStruct((EB,), f32),
        jax.ShapeDtypeStruct((EB,), f32),
    ],
    mesh=_MESH,
    compiler_params=_SC_PARAMS,
    scratch_types=[
        pltpu.VMEM((CB,), i32),
        pltpu.VMEM((CB,), i32),
        pltpu.VMEM((CB,), f32),
        pltpu.VMEM((CAPB + 16,), i32),
        pltpu.VMEM((CAPB + 16,), i32),
        pltpu.VMEM((CAPB + 16,), f32),
        pltpu.VMEM((CAPB + 16,), f32),
        pltpu.VMEM((CAPB + 16,), f32),
    ],
)
def _sc_bucket(src_h, dst_h, ea_h, srcp_o, dstp_o, eap_o, ekp_o, flagp_o,
               srcb, dstb, eab, so, do, eo, ko, fo):
    cid = lax.axis_index("c")
    sid = lax.axis_index("s")
    wid = sid * NC + cid

    def chunk_body(ci, cur):
        pltpu.sync_copy(src_h.at[pl.ds(ci * CB, CB)], srcb)
        pltpu.sync_copy(dst_h.at[pl.ds(ci * CB, CB)], dstb)
        pltpu.sync_copy(ea_h.at[pl.ds(ci * CB, CB)], eab)

        def vbody(v, cur):
            o = v * 16
            d16 = dstb[pl.ds(o, 16)]
            bkt = lax.shift_right_logical(d16 * 52429, 24)
            m = bkt == wid
            eid = ci * CB + o + lax.iota(i32, 16)
            ek16 = jnp.where(eid < E, 1.0, 0.0).astype(f32)
            fl16 = jnp.where((eid >= E) & (eid < E + NP), 1.0, 0.0).astype(f32)
            plsc.store_compressed(so.at[pl.ds(cur, 16)], srcb[pl.ds(o, 16)],
                                  mask=m)
            plsc.store_compressed(do.at[pl.ds(cur, 16)], d16, mask=m)
            plsc.store_compressed(eo.at[pl.ds(cur, 16)], eab[pl.ds(o, 16)],
                                  mask=m)
            plsc.store_compressed(ko.at[pl.ds(cur, 16)], ek16, mask=m)
            plsc.store_compressed(fo.at[pl.ds(cur, 16)], fl16, mask=m)
            return cur + jnp.max(plsc.all_reduce_population_count(m))

        return lax.fori_loop(0, CB // 16, vbody, cur)

    cur = lax.fori_loop(0, EB // CB, chunk_body, jnp.int32(0))

    def tail_body(t, _):
        o = t * 16
        pos = o + lax.iota(i32, 16)
        keep = pos < cur
        so[pl.ds(o, 16)] = jnp.where(keep, so[pl.ds(o, 16)], 0)
        do[pl.ds(o, 16)] = jnp.where(keep, do[pl.ds(o, 16)], wid * NB)
        eo[pl.ds(o, 16)] = jnp.where(keep, eo[pl.ds(o, 16)], 0.0)
        ko[pl.ds(o, 16)] = jnp.where(keep, ko[pl.ds(o, 16)], 0.0)
        fo[pl.ds(o, 16)] = jnp.where(keep, fo[pl.ds(o, 16)], 0.0)
        return 0

    lax.fori_loop(0, (CAPB + 16) // 16, tail_body, 0)

    base = wid * CAPB
    pltpu.sync_copy(so.at[pl.ds(0, CAPB)], srcp_o.at[pl.ds(base, CAPB)])
    pltpu.sync_copy(do.at[pl.ds(0, CAPB)], dstp_o.at[pl.ds(base, CAPB)])
    pltpu.sync_copy(eo.at[pl.ds(0, CAPB)], eap_o.at[pl.ds(base, CAPB)])
    pltpu.sync_copy(ko.at[pl.ds(0, CAPB)], ekp_o.at[pl.ds(base, CAPB)])
    pltpu.sync_copy(fo.at[pl.ds(0, CAPB)], flagp_o.at[pl.ds(base, CAPB)])


@functools.partial(
    pl.kernel,
    out_type=[
        jax.ShapeDtypeStruct((EB,), f32),
        jax.ShapeDtypeStruct((EB,), f32),
        jax.ShapeDtypeStruct((NC, NR, 128), f32),
        jax.ShapeDtypeStruct((NC, NR, 128), f32),
    ],
    mesh=_MESH,
    compiler_params=_SC_PARAMS,
    scratch_types=[
        pltpu.VMEM((NR, 128), f32),
        pltpu.VMEM((CB,), i32),
        pltpu.VMEM((CB,), i32),
        pltpu.VMEM((CB,), f32),
        pltpu.VMEM((CB,), f32),
        pltpu.VMEM((CB,), f32),
        pltpu.VMEM((CB,), f32),
        pltpu.VMEM((CB,), f32),
        pltpu.VMEM((NR, 128), f32),
        pltpu.VMEM((NR, 128), f32),
        pltpu.VMEM((NR, 128), f32),
        pltpu.VMEM((NR,), i32),
        pltpu.VMEM_SHARED((NR, 128), f32),
        pltpu.VMEM_SHARED((NR, 128), f32),
    ],
)
def _sc_edge_deg(src_h, dst_h, ea_h, ekp_h, flag_h, nk_h,
                 ek_o, w_o, degp_o, degsp_o,
                 nk_v, srcb, dstb, eab, ekpb, flb, ekob, wob,
                 dega, degsa, zbuf, rowidx, degsh, degssh):
    cid = lax.axis_index("c")
    sid = lax.axis_index("s")
    wid = sid * NC + cid

    pltpu.sync_copy(nk_h, nk_v)
    _zero_2d(dega, NR)
    _zero_2d(degsa, NR)
    _fill_rowidx(rowidx, NR)

    @pl.when(sid == 0)
    def _():
        _zero_2d(zbuf, NR)
        pltpu.sync_copy(zbuf, degsh)
        pltpu.sync_copy(zbuf, degssh)

    plsc.subcore_barrier()

    for ci in range(EWS // CB):
        base = wid * EWS + ci * CB
        pltpu.sync_copy(src_h.at[pl.ds(base, CB)], srcb)
        pltpu.sync_copy(dst_h.at[pl.ds(base, CB)], dstb)
        pltpu.sync_copy(ea_h.at[pl.ds(base, CB)], eab)
        pltpu.sync_copy(ekp_h.at[pl.ds(base, CB)], ekpb)
        pltpu.sync_copy(flag_h.at[pl.ds(base, CB)], flb)

        def body(j, _):
            o = j * 16
            s16 = srcb[pl.ds(o, 16)]
            d16 = dstb[pl.ds(o, 16)]
            sr, sc = _rc(s16)
            dr, dc = _rc(d16)
            nks = plsc.load_gather(nk_v, [sr, sc])
            nkd = plsc.load_gather(nk_v, [dr, dc])
            ek16 = ekpb[pl.ds(o, 16)] * nks * nkd
            w16 = eab[pl.ds(o, 16)] * ek16
            ekob[pl.ds(o, 16)] = ek16
            wob[pl.ds(o, 16)] = w16 + flb[pl.ds(o, 16)]
            plsc.addupdate_scatter(dega, [dr, dc], w16)
            plsc.addupdate_scatter(degsa, [dr, dc], ek16)
            return 0

        lax.fori_loop(0, CB // 16, body, 0)
        pltpu.sync_copy(ekob, ek_o.at[pl.ds(base, CB)])
        pltpu.sync_copy(wob, w_o.at[pl.ds(base, CB)])

    pltpu.sync_copy(dega, degsh.at[rowidx], add=True)
    pltpu.sync_copy(degsa, degssh.at[rowidx], add=True)
    plsc.subcore_barrier()

    @pl.when(sid == 0)
    def _():
        pltpu.sync_copy(degsh, degp_o.at[cid])
        pltpu.sync_copy(degssh, degsp_o.at[cid])


@functools.partial(
    pl.kernel,
    out_type=[jax.ShapeDtypeStruct((NP, 128), f32)],
    mesh=_MESH,
    compiler_params=_SC_PARAMS,
    scratch_types=[
        pltpu.VMEM((NR, 128), f32),
        pltpu.VMEM((JB, 128), i32),
        pltpu.VMEM((JB, 128), i32),
        pltpu.VMEM((JB, 128), f32),
        pltpu.VMEM((JB, 128), f32),
        pltpu.VMEM((128, 128), f32),
        pltpu.VMEM((128, 128), f32),
        pltpu.VMEM((NB, 128), f32),
        pltpu.SemaphoreType.DMA,
        pltpu.SemaphoreType.DMA,
    ],
)
def _sc_prop128(h_h, src2_h, dst2_h, w2_h, dis_h, p_o,
                dis_v, srcb, dstb, wb, normb, rows0, rows1, acc,
                semg0, semg1):
    cid = lax.axis_index("c")
    sid = lax.axis_index("s")
    wid = sid * NC + cid
    rows = (rows0, rows1)
    semg = (semg0, semg1)

    pltpu.sync_copy(dis_h, dis_v)
    _zero_2d(acc, NB)

    def blk_body(blk, _):
        rowbase = pl.multiple_of(wid * (CAPB // 128) + blk * JB, 8)
        pltpu.sync_copy(src2_h.at[pl.ds(rowbase, JB)], srcb)
        pltpu.sync_copy(dst2_h.at[pl.ds(rowbase, JB)], dstb)
        pltpu.sync_copy(w2_h.at[pl.ds(rowbase, JB)], wb)

        dg0 = pltpu.async_copy(h_h.at[srcb.at[0]], rows[0], semg[0])

        def norm_body(it, _):
            j = it >> 3
            o = (it & 7) * 16
            s16 = srcb[j, pl.ds(o, 16)]
            d16 = dstb[j, pl.ds(o, 16)]
            sr, sc = _rc(s16)
            dr, dc = _rc(d16)
            ds_ = plsc.load_gather(dis_v, [sr, sc])
            dd_ = plsc.load_gather(dis_v, [dr, dc])
            normb[j, pl.ds(o, 16)] = wb[j, pl.ds(o, 16)] * ds_ * dd_
            dstb[j, pl.ds(o, 16)] = d16 - wid * NB
            return 0

        lax.fori_loop(0, JB * 8, norm_body, 0)

        dg = [dg0, None]
        for j in range(JB):
            p = j & 1
            q = (j + 1) & 1
            dg[p].wait()
            if j + 1 < JB:
                dg[q] = pltpu.async_copy(
                    h_h.at[srcb.at[j + 1]], rows[q], semg[q])

            colidx = lax.iota(i32, 16)

            def scale_acc(r, _):
                jr = jnp.full((16,), j, i32)
                rr = jnp.full((16,), 0, i32) + r
                wspl = plsc.load_gather(normb, [jr, rr])
                dspl = plsc.load_gather(dstb, [jr, rr])
                for c in range(8):
                    v = rows[p][r, pl.ds(c * 16, 16)] * wspl
                    plsc.addupdate_scatter(
                        acc, [dspl, colidx + c * 16], v)
                return 0

            lax.fori_loop(0, 128, scale_acc, 0)
        return 0

    lax.fori_loop(0, CAPB // (JB * 128), blk_body, 0)
    pltpu.sync_copy(acc, p_o.at[pl.ds(wid * NB, NB)])


@functools.partial(
    pl.kernel,
    out_type=[jax.ShapeDtypeStruct((NC, NR, 128), f32)],
    mesh=_MESH,
    compiler_params=_SC_PARAMS,
    scratch_types=[
        pltpu.VMEM((NR, 128), f32),
        pltpu.VMEM((CB,), i32),
        pltpu.VMEM((CB,), i32),
        pltpu.VMEM((CB,), f32),
        pltpu.VMEM((NR, 128), f32),
        pltpu.VMEM((NR, 128), f32),
        pltpu.VMEM((NR,), i32),
        pltpu.VMEM_SHARED((NR, 128), f32),
    ],
)
def _sc_prop_scalar(hs_h, src_h, dst_h, ek_h, s0_o,
                    hs_v, srcb, dstb, ekb, acc, zbuf, rowidx, accsh):
    cid = lax.axis_index("c")
    sid = lax.axis_index("s")
    wid = sid * NC + cid

    pltpu.sync_copy(hs_h, hs_v)
    _zero_2d(acc, NR)
    _fill_rowidx(rowidx, NR)

    @pl.when(sid == 0)
    def _():
        _zero_2d(zbuf, NR)
        pltpu.sync_copy(zbuf, accsh)

    plsc.subcore_barrier()

    for ci in range(EWS // CB):
        base = wid * EWS + ci * CB
        pltpu.sync_copy(src_h.at[pl.ds(base, CB)], srcb)
        pltpu.sync_copy(dst_h.at[pl.ds(base, CB)], dstb)
        pltpu.sync_copy(ek_h.at[pl.ds(base, CB)], ekb)

        def body(j, _):
            o = j * 16
            s16 = srcb[pl.ds(o, 16)]
            d16 = dstb[pl.ds(o, 16)]
            sr, sc = _rc(s16)
            dr, dc = _rc(d16)
            v16 = plsc.load_gather(hs_v, [sr, sc]) * ekb[pl.ds(o, 16)]
            plsc.addupdate_scatter(acc, [dr, dc], v16)
            return 0

        lax.fori_loop(0, CB // 16, body, 0)

    pltpu.sync_copy(acc, accsh.at[rowidx], add=True)
    plsc.subcore_barrier()

    @pl.when(sid == 0)
    def _():
        pltpu.sync_copy(accsh, s0_o.at[cid])


def _tca_body(h_ref, w_ref, degp_ref, degsp_ref,
              hh_ref, dis_ref, diss_ref, invds_ref):
    deg = 1.0 + degp_ref[0] + degp_ref[1]
    degs = 1.0 + degsp_ref[0] + degsp_ref[1]
    dis_ref[...] = 1.0 / jnp.sqrt(deg)
    diss = 1.0 / jnp.sqrt(degs)
    diss_ref[...] = diss
    invds_ref[...] = diss * diss
    hh_ref[...] = jnp.dot(h_ref[...], w_ref[...], preferred_element_type=f32)


def _tca(h, W, degp, degsp):
    grid = NP // 1024
    return pl.pallas_call(
        _tca_body,
        grid=(grid,),
        in_specs=[
            pl.BlockSpec((1024, F), lambda i: (i, 0)),
            pl.BlockSpec((F, F), lambda i: (0, 0)),
            pl.BlockSpec((NC, 8, 128), lambda i: (0, i, 0)),
            pl.BlockSpec((NC, 8, 128), lambda i: (0, i, 0)),
        ],
        out_specs=[
            pl.BlockSpec((1024, F), lambda i: (i, 0)),
            pl.BlockSpec((8, 128), lambda i: (i, 0)),
            pl.BlockSpec((8, 128), lambda i: (i, 0)),
            pl.BlockSpec((8, 128), lambda i: (i, 0)),
        ],
        out_shape=[
            jax.ShapeDtypeStruct((NP, F), f32),
            jax.ShapeDtypeStruct((NR, 128), f32),
            jax.ShapeDtypeStruct((NR, 128), f32),
            jax.ShapeDtypeStruct((NR, 128), f32),
        ],
    )(h, W, degp, degsp)


def _tcb_body(p0_ref, b_ref, ws3_ref, diss_ref, hl_ref, hs_ref, hss_ref):
    hl = jnp.maximum(p0_ref[...] + b_ref[...], 0.0)
    hl_ref[...] = hl
    hl3 = hl.reshape(8, 128, F)
    hs = jnp.sum(hl3 * ws3_ref[...], axis=2)
    hs_ref[...] = hs
    hss_ref[...] = hs * diss_ref[...]


def _tcb(P0, b, Ws3, dis_s):
    grid = NP // 1024
    return pl.pallas_call(
        _tcb_body,
        grid=(grid,),
        in_specs=[
            pl.BlockSpec((1024, F), lambda i: (i, 0)),
            pl.BlockSpec((1, F), lambda i: (0, 0)),
            pl.BlockSpec((1, 1, F), lambda i: (0, 0, 0)),
            pl.BlockSpec((8, 128), lambda i: (i, 0)),
        ],
        out_specs=[
            pl.BlockSpec((1024, F), lambda i: (i, 0)),
            pl.BlockSpec((8, 128), lambda i: (i, 0)),
            pl.BlockSpec((8, 128), lambda i: (i, 0)),
        ],
        out_shape=[
            jax.ShapeDtypeStruct((NP, F), f32),
            jax.ShapeDtypeStruct((NR, 128), f32),
            jax.ShapeDtypeStruct((NR, 128), f32),
        ],
    )(P0, b, Ws3, dis_s)


def _tcs_body(s0_ref, hs_ref, diss_ref, invds_ref, bs_ref, sc_ref):
    sc_ref[...] = (diss_ref[...] * (s0_ref[0] + s0_ref[1])
                   + hs_ref[...] * invds_ref[...] + bs_ref[0, 0])


def _tcs(S0, hs, dis_s, invdeg_s, bs2):
    return pl.pallas_call(
        _tcs_body,
        out_shape=jax.ShapeDtypeStruct((NR, 128), f32),
    )(S0, hs, dis_s, invdeg_s, bs2)


def _tr(v):
    return jnp.swapaxes(v, 0, 1)


def _tcc_body(scr_ref, batr_ref, batfr_ref, nkr_ref, hl_ref,
              nkn_ref, hp_ref, x_ref, mx_ref, kc_ref, cnt_ref):
    gids_r = lax.broadcasted_iota(i32, (1, G), 1)
    gids_c = lax.broadcasted_iota(i32, (G, 1), 0).astype(f32)
    oh = (gids_c == batfr_ref[...]).astype(f32)

    mx_ref[...] = jnp.full((G, F), -1e30, f32)
    kc_ref[...] = jnp.zeros((1, G), f32)
    cnt_ref[...] = jnp.zeros((1, G), f32)

    def pa_body(rb, _):
        r0 = rb * 128
        bat_col = _tr(batr_ref[:, pl.ds(r0, 128)])
        ohc_blk = (bat_col == gids_r).astype(f32)
        nk_row = nkr_ref[:, pl.ds(r0, 128)]
        kc_ref[...] += jnp.dot(nk_row, ohc_blk,
                               preferred_element_type=f32)
        return 0

    lax.fori_loop(0, NP // 128, pa_body, 0)
    k_col = _tr(jnp.ceil(0.5 * kc_ref[...]))

    def pb_body(rb, _):
        r0 = rb * 128
        sc_row = scr_ref[:, pl.ds(r0, 128)]
        bat_row = batr_ref[:, pl.ds(r0, 128)]
        nk_row = nkr_ref[:, pl.ds(r0, 128)]
        sc_col = _tr(sc_row)
        bat_col = _tr(bat_row)
        nk_col = _tr(nk_row)
        ohc_blk = (bat_col == gids_r).astype(f32)
        kp_col = jnp.dot(ohc_blk, k_col, preferred_element_type=f32)
        ridx = lax.broadcasted_iota(i32, (128, 1), 0) + r0

        def col_body(cb, acc):
            c0 = cb * 1024
            sc_c = scr_ref[:, pl.ds(c0, 1024)]
            bat_c = batr_ref[:, pl.ds(c0, 1024)]
            keep_c = nkr_ref[:, pl.ds(c0, 1024)]
            cidx = lax.broadcasted_iota(i32, (1, 1024), 1) + c0
            before = (sc_c > sc_col) | ((sc_c == sc_col) & (cidx < ridx))
            cmp = ((bat_c == bat_col) & (keep_c > 0.0) & before)
            return acc + jnp.sum(cmp.astype(f32), axis=1, keepdims=True)

        rank = lax.fori_loop(0, NP // 1024, col_body,
                             jnp.zeros((128, 1), f32))
        nkn_col = nk_col * (rank < kp_col).astype(f32)
        cnt_ref[...] += jnp.dot(_tr(nkn_col), ohc_blk,
                                preferred_element_type=f32)
        nkn_ref[pl.ds(rb, 1), :] = _tr(nkn_col)
        hp_blk = hl_ref[pl.ds(r0, 128), :] * (jnp.tanh(sc_col) * nkn_col)
        hp_ref[pl.ds(r0, 128), :] = hp_blk

        g_lo = jnp.min(bat_row)
        g_hi = jnp.max(bat_row)

        def g_body(g, _):
            m = (bat_col == g) & (nkn_col > 0.0)
            vals = jnp.where(m, hp_blk, jnp.full((128, F), -1e30, f32))
            mrow = jnp.max(vals, axis=0, keepdims=True)
            cur = mx_ref[pl.ds(g, 1), :]
            mx_ref[pl.ds(g, 1), :] = jnp.maximum(cur, mrow)
            return 0

        lax.fori_loop(g_lo, g_hi + 1, g_body, 0)
        return 0

    lax.fori_loop(0, NP // 128, pb_body, 0)

    cnt = _tr(cnt_ref[...])
    seg_sum = jnp.dot(oh, hp_ref[...], preferred_element_type=f32)
    mean = seg_sum / jnp.maximum(cnt, 1.0)
    mx = jnp.where(cnt > 0.0, mx_ref[...], jnp.zeros((G, F), f32))
    x_ref[:, 0:F] = mx
    x_ref[:, F:2 * F] = mean


def _tcc(score_r, batr, batfr, nkr, hl):
    return pl.pallas_call(
        _tcc_body,
        out_shape=[
            jax.ShapeDtypeStruct((NR, 128), f32),
            jax.ShapeDtypeStruct((NP, F), f32),
            jax.ShapeDtypeStruct((G, 2 * F), f32),
        ],
        scratch_shapes=[
            pltpu.VMEM((G, F), f32),
            pltpu.VMEM((1, G), f32),
            pltpu.VMEM((1, G), f32),
        ],
    )(score_r, batr, batfr, nkr, hl)


def _tcd_body(x1_ref, x2_ref, x3_ref, l1_ref, l1b_ref, l2_ref, l2b_ref,
              l3_ref, l3b_ref, o_ref):
    z = x1_ref[...] + x2_ref[...] + x3_ref[...]
    z = jnp.maximum(
        jnp.dot(z, l1_ref[...], preferred_element_type=f32) + l1b_ref[...],
        0.0)
    z = jnp.maximum(
        jnp.dot(z, l2_ref[...], preferred_element_type=f32) + l2b_ref[...],
        0.0)
    z = jnp.dot(z, l3_ref[...], preferred_element_type=f32) + l3b_ref[...]
    m = jnp.max(z, axis=1, keepdims=True)
    lse = m + jnp.log(jnp.sum(jnp.exp(z - m), axis=1, keepdims=True))
    o_ref[...] = z - lse


def _tcd(x1, x2, x3, L1, l1b, L2, l2b, L3, l3b):
    return pl.pallas_call(
        _tcd_body,
        out_shape=jax.ShapeDtypeStruct((G, 2), f32),
    )(x1, x2, x3, L1, l1b.reshape(1, -1), L2, l2b.reshape(1, -1),
      L3, l3b.reshape(1, -1))


def kernel(x, edge_index, edge_attr, batch, W1, b1, Ws1, bs1, W2, b2, Ws2,
           bs2, W3, b3, Ws3, bs3, L1, l1b, L2, l2b, L3, l3b):
    loops = jnp.arange(NP, dtype=jnp.int32)
    npad = EB - E - NP
    src0 = jnp.concatenate([edge_index[0], loops,
                            jnp.zeros((npad,), jnp.int32)])
    dst0 = jnp.concatenate([edge_index[1], loops,
                            jnp.full((npad,), DPAD, jnp.int32)])
    ea0 = jnp.concatenate([edge_attr, jnp.zeros((NP + npad,), f32)])

    srcp, dstp, eap, ek, flagp = _sc_bucket(src0, dst0, ea0)
    src2 = srcp.reshape(EB // 128, 128)
    dst2 = dstp.reshape(EB // 128, 128)

    bat = jnp.concatenate([batch, jnp.full((NP - N,), G - 1, jnp.int32)])
    batr = bat.reshape(1, NP)
    batfr = bat.astype(f32).reshape(1, NP)
    h = jnp.concatenate([x, jnp.zeros((NP - N, F), f32)])
    nk = jnp.ones((NR, 128), f32)

    params = [(W1, b1, Ws1, bs1), (W2, b2, Ws2, bs2), (W3, b3, Ws3, bs3)]
    xs = []
    for (W, b, Ws, bs) in params:
        ek, w, degP, degsP = _sc_edge_deg(srcp, dstp, eap, ek, flagp, nk)
        H, dis, dis_s, invdeg_s = _tca(h, W, degP, degsP)
        (P0,) = _sc_prop128(H, src2, dst2, w.reshape(EB // 128, 128), dis)
        hl, hs, hss = _tcb(P0, b.reshape(1, F), Ws.reshape(1, 1, F), dis_s)
        (S0,) = _sc_prop_scalar(hss, srcp, dstp, ek)
        score = _tcs(S0, hs, dis_s, invdeg_s, bs.reshape(1, 1))
        nk, hp, xl = _tcc(score.reshape(1, NP), batr, batfr,
                          nk.reshape(1, NP), hl)
        h = hp
        xs.append(xl)

    return _tcd(xs[0], xs[1], xs[2], L1, l1b, L2, l2b, L3, l3b)


# speedup vs baseline: 1.0299x; 1.0299x over previous
import functools

import jax
import jax.numpy as jnp
from jax import lax
from jax.experimental import pallas as pl
from jax.experimental.pallas import tpu as pltpu
from jax.experimental.pallas import tpu_sc as plsc

f32 = jnp.float32
i32 = jnp.int32

N = 10000
E = 320000
G = 64
F = 128
NP = 10240
NC = 2
NS = 16
NW = NC * NS
NB = NP // NW
CAPB = 12288
EB = NW * CAPB
EWS = EB // NW
CB = 2048
NR = NP // 128
JB = 8
DPAD = 16384

_MESH = plsc.VectorSubcoreMesh(
    core_axis_name="c", subcore_axis_name="s", num_cores=NC, num_subcores=NS)
_SC_PARAMS = pltpu.CompilerParams(needs_layout_passes=False)


def _zero16():
    return jnp.zeros((16,), f32)


def _zero_2d(ref, nrows):
    def body(it, _):
        r = it >> 3
        c = (it & 7) * 16
        ref[r, pl.ds(c, 16)] = _zero16()
        return 0
    lax.fori_loop(0, nrows * 8, body, 0)


def _fill_rowidx(ref, nrows):
    def body(it, _):
        ref[pl.ds(it * 16, 16)] = lax.iota(i32, 16) + it * 16
        return 0
    lax.fori_loop(0, nrows // 16, body, 0)


def _rc(v16):
    return lax.shift_right_logical(v16, 7), lax.bitwise_and(v16, 127)


@functools.partial(
    pl.kernel,
    out_type=[
        jax.ShapeDtypeStruct((EB,), i32),
        jax.ShapeDtypeStruct((EB,), i32),
        jax.ShapeDtypeStruct((EB,), f32),
        jax.ShapeDtypeStruct((EB,), f32),
        jax.ShapeDtypeStruct((EB,), f32),
    ],
    mesh=_MESH,
    compiler_params=_SC_PARAMS,
    scratch_types=[
        pltpu.VMEM((CB,), i32),
        pltpu.VMEM((CB,), i32),
        pltpu.VMEM((CB,), f32),
        pltpu.VMEM((CAPB + 16,), i32),
        pltpu.VMEM((CAPB + 16,), i32),
        pltpu.VMEM((CAPB + 16,), f32),
        pltpu.VMEM((CAPB + 16,), f32),
        pltpu.VMEM((CAPB + 16,), f32),
    ],
)
def _sc_bucket(src_h, dst_h, ea_h, srcp_o, dstp_o, eap_o, ekp_o, flagp_o,
               srcb, dstb, eab, so, do, eo, ko, fo):
    cid = lax.axis_index("c")
    sid = lax.axis_index("s")
    wid = sid * NC + cid

    def chunk_body(ci, cur):
        pltpu.sync_copy(src_h.at[pl.ds(ci * CB, CB)], srcb)
        pltpu.sync_copy(dst_h.at[pl.ds(ci * CB, CB)], dstb)
        pltpu.sync_copy(ea_h.at[pl.ds(ci * CB, CB)], eab)

        def vbody(v, cur):
            o = v * 16
            d16 = dstb[pl.ds(o, 16)]
            bkt = lax.shift_right_logical(d16 * 52429, 24)
            m = bkt == wid
            eid = ci * CB + o + lax.iota(i32, 16)
            ek16 = jnp.where(eid < E, 1.0, 0.0).astype(f32)
            fl16 = jnp.where((eid >= E) & (eid < E + NP), 1.0, 0.0).astype(f32)
            plsc.store_compressed(so.at[pl.ds(cur, 16)], srcb[pl.ds(o, 16)],
                                  mask=m)
            plsc.store_compressed(do.at[pl.ds(cur, 16)], d16, mask=m)
            plsc.store_compressed(eo.at[pl.ds(cur, 16)], eab[pl.ds(o, 16)],
                                  mask=m)
            plsc.store_compressed(ko.at[pl.ds(cur, 16)], ek16, mask=m)
            plsc.store_compressed(fo.at[pl.ds(cur, 16)], fl16, mask=m)
            return cur + jnp.max(plsc.all_reduce_population_count(m))

        return lax.fori_loop(0, CB // 16, vbody, cur)

    cur = lax.fori_loop(0, EB // CB, chunk_body, jnp.int32(0))

    def tail_body(t, _):
        o = t * 16
        pos = o + lax.iota(i32, 16)
        keep = pos < cur
        so[pl.ds(o, 16)] = jnp.where(keep, so[pl.ds(o, 16)], 0)
        do[pl.ds(o, 16)] = jnp.where(keep, do[pl.ds(o, 16)], wid * NB)
        eo[pl.ds(o, 16)] = jnp.where(keep, eo[pl.ds(o, 16)], 0.0)
        ko[pl.ds(o, 16)] = jnp.where(keep, ko[pl.ds(o, 16)], 0.0)
        fo[pl.ds(o, 16)] = jnp.where(keep, fo[pl.ds(o, 16)], 0.0)
        return 0

    lax.fori_loop(0, (CAPB + 16) // 16, tail_body, 0)

    base = wid * CAPB
    pltpu.sync_copy(so.at[pl.ds(0, CAPB)], srcp_o.at[pl.ds(base, CAPB)])
    pltpu.sync_copy(do.at[pl.ds(0, CAPB)], dstp_o.at[pl.ds(base, CAPB)])
    pltpu.sync_copy(eo.at[pl.ds(0, CAPB)], eap_o.at[pl.ds(base, CAPB)])
    pltpu.sync_copy(ko.at[pl.ds(0, CAPB)], ekp_o.at[pl.ds(base, CAPB)])
    pltpu.sync_copy(fo.at[pl.ds(0, CAPB)], flagp_o.at[pl.ds(base, CAPB)])


@functools.partial(
    pl.kernel,
    out_type=[
        jax.ShapeDtypeStruct((EB,), f32),
        jax.ShapeDtypeStruct((EB,), f32),
        jax.ShapeDtypeStruct((NC, NR, 128), f32),
        jax.ShapeDtypeStruct((NC, NR, 128), f32),
    ],
    mesh=_MESH,
    compiler_params=_SC_PARAMS,
    scratch_types=[
        pltpu.VMEM((NR, 128), f32),
        pltpu.VMEM((CB,), i32),
        pltpu.VMEM((CB,), i32),
        pltpu.VMEM((CB,), f32),
        pltpu.VMEM((CB,), f32),
        pltpu.VMEM((CB,), f32),
        pltpu.VMEM((CB,), f32),
        pltpu.VMEM((CB,), f32),
        pltpu.VMEM((NR, 128), f32),
        pltpu.VMEM((NR, 128), f32),
        pltpu.VMEM((NR, 128), f32),
        pltpu.VMEM((NR,), i32),
        pltpu.VMEM_SHARED((NR, 128), f32),
        pltpu.VMEM_SHARED((NR, 128), f32),
    ],
)
def _sc_edge_deg(src_h, dst_h, ea_h, ekp_h, flag_h, nk_h,
                 ek_o, w_o, degp_o, degsp_o,
                 nk_v, srcb, dstb, eab, ekpb, flb, ekob, wob,
                 dega, degsa, zbuf, rowidx, degsh, degssh):
    cid = lax.axis_index("c")
    sid = lax.axis_index("s")
    wid = sid * NC + cid

    pltpu.sync_copy(nk_h, nk_v)
    _zero_2d(dega, NR)
    _zero_2d(degsa, NR)
    _fill_rowidx(rowidx, NR)

    @pl.when(sid == 0)
    def _():
        _zero_2d(zbuf, NR)
        pltpu.sync_copy(zbuf, degsh)
        pltpu.sync_copy(zbuf, degssh)

    plsc.subcore_barrier()

    for ci in range(EWS // CB):
        base = wid * EWS + ci * CB
        pltpu.sync_copy(src_h.at[pl.ds(base, CB)], srcb)
        pltpu.sync_copy(dst_h.at[pl.ds(base, CB)], dstb)
        pltpu.sync_copy(ea_h.at[pl.ds(base, CB)], eab)
        pltpu.sync_copy(ekp_h.at[pl.ds(base, CB)], ekpb)
        pltpu.sync_copy(flag_h.at[pl.ds(base, CB)], flb)

        def body(j, _):
            o = j * 16
            s16 = srcb[pl.ds(o, 16)]
            d16 = dstb[pl.ds(o, 16)]
            sr, sc = _rc(s16)
            dr, dc = _rc(d16)
            nks = plsc.load_gather(nk_v, [sr, sc])
            nkd = plsc.load_gather(nk_v, [dr, dc])
            ek16 = ekpb[pl.ds(o, 16)] * nks * nkd
            w16 = eab[pl.ds(o, 16)] * ek16
            ekob[pl.ds(o, 16)] = ek16
            wob[pl.ds(o, 16)] = w16 + flb[pl.ds(o, 16)]
            plsc.addupdate_scatter(dega, [dr, dc], w16)
            plsc.addupdate_scatter(degsa, [dr, dc], ek16)
            return 0

        lax.fori_loop(0, CB // 16, body, 0)
        pltpu.sync_copy(ekob, ek_o.at[pl.ds(base, CB)])
        pltpu.sync_copy(wob, w_o.at[pl.ds(base, CB)])

    pltpu.sync_copy(dega, degsh.at[rowidx], add=True)
    pltpu.sync_copy(degsa, degssh.at[rowidx], add=True)
    plsc.subcore_barrier()

    @pl.when(sid == 0)
    def _():
        pltpu.sync_copy(degsh, degp_o.at[cid])
        pltpu.sync_copy(degssh, degsp_o.at[cid])


@functools.partial(
    pl.kernel,
    out_type=[jax.ShapeDtypeStruct((NP, 128), f32)],
    mesh=_MESH,
    compiler_params=_SC_PARAMS,
    scratch_types=[
        pltpu.VMEM((NR, 128), f32),
        pltpu.VMEM((JB, 128), i32),
        pltpu.VMEM((JB, 128), i32),
        pltpu.VMEM((JB, 128), f32),
        pltpu.VMEM((JB, 128), f32),
        pltpu.VMEM((128, 128), f32),
        pltpu.VMEM((128, 128), f32),
        pltpu.VMEM_SHARED((NS * NB, 128), f32),
        pltpu.SemaphoreType.DMA,
        pltpu.SemaphoreType.DMA,
    ],
)
def _sc_prop128(h_h, src2_h, dst2_h, w2_h, dis_h, p_o,
                dis_v, srcb, dstb, wb, normb, rows0, rows1, acc,
                semg0, semg1):
    cid = lax.axis_index("c")
    sid = lax.axis_index("s")
    wid = sid * NC + cid
    rows = (rows0, rows1)
    semg = (semg0, semg1)

    pltpu.sync_copy(dis_h, dis_v)
    _zero_2d(rows0, 128)
    for q in range(NB // 128):
        pltpu.sync_copy(rows0, acc.at[pl.ds(sid * NB + q * 128, 128)])
    plsc.subcore_barrier()

    def blk_body(blk, _):
        rowbase = pl.multiple_of(wid * (CAPB // 128) + blk * JB, 8)
        pltpu.sync_copy(src2_h.at[pl.ds(rowbase, JB)], srcb)
        pltpu.sync_copy(dst2_h.at[pl.ds(rowbase, JB)], dstb)
        pltpu.sync_copy(w2_h.at[pl.ds(rowbase, JB)], wb)

        dg0 = pltpu.async_copy(h_h.at[srcb.at[0]], rows[0], semg[0])

        def norm_body(it, _):
            j = it >> 3
            o = (it & 7) * 16
            s16 = srcb[j, pl.ds(o, 16)]
            d16 = dstb[j, pl.ds(o, 16)]
            sr, sc = _rc(s16)
            dr, dc = _rc(d16)
            ds_ = plsc.load_gather(dis_v, [sr, sc])
            dd_ = plsc.load_gather(dis_v, [dr, dc])
            normb[j, pl.ds(o, 16)] = wb[j, pl.ds(o, 16)] * ds_ * dd_
            dstb[j, pl.ds(o, 16)] = d16 - (wid - sid) * NB
            return 0

        lax.fori_loop(0, JB * 8, norm_body, 0)

        dg = [dg0, None]
        for j in range(JB):
            p = j & 1
            q = (j + 1) & 1
            dg[p].wait()
            if j + 1 < JB:
                dg[q] = pltpu.async_copy(
                    h_h.at[srcb.at[j + 1]], rows[q], semg[q])

            def scale(r, _):
                jr = jnp.full((16,), j, i32)
                rr = jnp.full((16,), 0, i32) + r
                wspl = plsc.load_gather(normb, [jr, rr])
                for c in range(8):
                    rows[p][r, pl.ds(c * 16, 16)] = (
                        rows[p][r, pl.ds(c * 16, 16)] * wspl)
                return 0

            lax.fori_loop(0, 128, scale, 0)
            pltpu.sync_copy(rows[p], acc.at[dstb.at[j]], add=True)
        return 0

    lax.fori_loop(0, CAPB // (JB * 128), blk_body, 0)
    pltpu.sync_copy(acc.at[pl.ds(sid * NB, NB)],
                    p_o.at[pl.ds(wid * NB, NB)])


@functools.partial(
    pl.kernel,
    out_type=[jax.ShapeDtypeStruct((NC, NR, 128), f32)],
    mesh=_MESH,
    compiler_params=_SC_PARAMS,
    scratch_types=[
        pltpu.VMEM((NR, 128), f32),
        pltpu.VMEM((CB,), i32),
        pltpu.VMEM((CB,), i32),
        pltpu.VMEM((CB,), f32),
        pltpu.VMEM((NR, 128), f32),
        pltpu.VMEM((NR, 128), f32),
        pltpu.VMEM((NR,), i32),
        pltpu.VMEM_SHARED((NR, 128), f32),
    ],
)
def _sc_prop_scalar(hs_h, src_h, dst_h, ek_h, s0_o,
                    hs_v, srcb, dstb, ekb, acc, zbuf, rowidx, accsh):
    cid = lax.axis_index("c")
    sid = lax.axis_index("s")
    wid = sid * NC + cid

    pltpu.sync_copy(hs_h, hs_v)
    _zero_2d(acc, NR)
    _fill_rowidx(rowidx, NR)

    @pl.when(sid == 0)
    def _():
        _zero_2d(zbuf, NR)
        pltpu.sync_copy(zbuf, accsh)

    plsc.subcore_barrier()

    for ci in range(EWS // CB):
        base = wid * EWS + ci * CB
        pltpu.sync_copy(src_h.at[pl.ds(base, CB)], srcb)
        pltpu.sync_copy(dst_h.at[pl.ds(base, CB)], dstb)
        pltpu.sync_copy(ek_h.at[pl.ds(base, CB)], ekb)

        def body(j, _):
            o = j * 16
            s16 = srcb[pl.ds(o, 16)]
            d16 = dstb[pl.ds(o, 16)]
            sr, sc = _rc(s16)
            dr, dc = _rc(d16)
            v16 = plsc.load_gather(hs_v, [sr, sc]) * ekb[pl.ds(o, 16)]
            plsc.addupdate_scatter(acc, [dr, dc], v16)
            return 0

        lax.fori_loop(0, CB // 16, body, 0)

    pltpu.sync_copy(acc, accsh.at[rowidx], add=True)
    plsc.subcore_barrier()

    @pl.when(sid == 0)
    def _():
        pltpu.sync_copy(accsh, s0_o.at[cid])


def _tca_body(h_ref, w_ref, degp_ref, degsp_ref,
              hh_ref, dis_ref, diss_ref, invds_ref):
    deg = 1.0 + degp_ref[0] + degp_ref[1]
    degs = 1.0 + degsp_ref[0] + degsp_ref[1]
    dis_ref[...] = 1.0 / jnp.sqrt(deg)
    diss = 1.0 / jnp.sqrt(degs)
    diss_ref[...] = diss
    invds_ref[...] = diss * diss
    hh_ref[...] = jnp.dot(h_ref[...], w_ref[...], preferred_element_type=f32)


def _tca(h, W, degp, degsp):
    grid = NP // 1024
    return pl.pallas_call(
        _tca_body,
        grid=(grid,),
        in_specs=[
            pl.BlockSpec((1024, F), lambda i: (i, 0)),
            pl.BlockSpec((F, F), lambda i: (0, 0)),
            pl.BlockSpec((NC, 8, 128), lambda i: (0, i, 0)),
            pl.BlockSpec((NC, 8, 128), lambda i: (0, i, 0)),
        ],
        out_specs=[
            pl.BlockSpec((1024, F), lambda i: (i, 0)),
            pl.BlockSpec((8, 128), lambda i: (i, 0)),
            pl.BlockSpec((8, 128), lambda i: (i, 0)),
            pl.BlockSpec((8, 128), lambda i: (i, 0)),
        ],
        out_shape=[
            jax.ShapeDtypeStruct((NP, F), f32),
            jax.ShapeDtypeStruct((NR, 128), f32),
            jax.ShapeDtypeStruct((NR, 128), f32),
            jax.ShapeDtypeStruct((NR, 128), f32),
        ],
    )(h, W, degp, degsp)


def _tcb_body(p0_ref, b_ref, ws3_ref, diss_ref, hl_ref, hs_ref, hss_ref):
    hl = jnp.maximum(p0_ref[...] + b_ref[...], 0.0)
    hl_ref[...] = hl
    hl3 = hl.reshape(8, 128, F)
    hs = jnp.sum(hl3 * ws3_ref[...], axis=2)
    hs_ref[...] = hs
    hss_ref[...] = hs * diss_ref[...]


def _tcb(P0, b, Ws3, dis_s):
    grid = NP // 1024
    return pl.pallas_call(
        _tcb_body,
        grid=(grid,),
        in_specs=[
            pl.BlockSpec((1024, F), lambda i: (i, 0)),
            pl.BlockSpec((1, F), lambda i: (0, 0)),
            pl.BlockSpec((1, 1, F), lambda i: (0, 0, 0)),
            pl.BlockSpec((8, 128), lambda i: (i, 0)),
        ],
        out_specs=[
            pl.BlockSpec((1024, F), lambda i: (i, 0)),
            pl.BlockSpec((8, 128), lambda i: (i, 0)),
            pl.BlockSpec((8, 128), lambda i: (i, 0)),
        ],
        out_shape=[
            jax.ShapeDtypeStruct((NP, F), f32),
            jax.ShapeDtypeStruct((NR, 128), f32),
            jax.ShapeDtypeStruct((NR, 128), f32),
        ],
    )(P0, b, Ws3, dis_s)


def _tcs_body(s0_ref, hs_ref, diss_ref, invds_ref, bs_ref, sc_ref):
    sc_ref[...] = (diss_ref[...] * (s0_ref[0] + s0_ref[1])
                   + hs_ref[...] * invds_ref[...] + bs_ref[0, 0])


def _tcs(S0, hs, dis_s, invdeg_s, bs2):
    return pl.pallas_call(
        _tcs_body,
        out_shape=jax.ShapeDtypeStruct((NR, 128), f32),
    )(S0, hs, dis_s, invdeg_s, bs2)


def _tr(v):
    return jnp.swapaxes(v, 0, 1)


def _tcc_body(scr_ref, batr_ref, batfr_ref, nkr_ref, hl_ref,
              nkn_ref, hp_ref, x_ref, mx_ref, kc_ref, cnt_ref):
    gids_r = lax.broadcasted_iota(i32, (1, G), 1)
    gids_c = lax.broadcasted_iota(i32, (G, 1), 0).astype(f32)
    oh = (gids_c == batfr_ref[...]).astype(f32)

    mx_ref[...] = jnp.full((G, F), -1e30, f32)
    kc_ref[...] = jnp.zeros((1, G), f32)
    cnt_ref[...] = jnp.zeros((1, G), f32)

    def pa_body(rb, _):
        r0 = rb * 128
        bat_col = _tr(batr_ref[:, pl.ds(r0, 128)])
        ohc_blk = (bat_col == gids_r).astype(f32)
        nk_row = nkr_ref[:, pl.ds(r0, 128)]
        kc_ref[...] += jnp.dot(nk_row, ohc_blk,
                               preferred_element_type=f32)
        return 0

    lax.fori_loop(0, NP // 128, pa_body, 0)
    k_col = _tr(jnp.ceil(0.5 * kc_ref[...]))

    def pb_body(rb, _):
        r0 = rb * 128
        sc_row = scr_ref[:, pl.ds(r0, 128)]
        bat_row = batr_ref[:, pl.ds(r0, 128)]
        nk_row = nkr_ref[:, pl.ds(r0, 128)]
        sc_col = _tr(sc_row)
        bat_col = _tr(bat_row)
        nk_col = _tr(nk_row)
        ohc_blk = (bat_col == gids_r).astype(f32)
        kp_col = jnp.dot(ohc_blk, k_col, preferred_element_type=f32)
        ridx = lax.broadcasted_iota(i32, (128, 1), 0) + r0

        def col_body(cb, acc):
            c0 = cb * 1024
            sc_c = scr_ref[:, pl.ds(c0, 1024)]
            bat_c = batr_ref[:, pl.ds(c0, 1024)]
            keep_c = nkr_ref[:, pl.ds(c0, 1024)]
            cidx = lax.broadcasted_iota(i32, (1, 1024), 1) + c0
            before = (sc_c > sc_col) | ((sc_c == sc_col) & (cidx < ridx))
            cmp = ((bat_c == bat_col) & (keep_c > 0.0) & before)
            return acc + jnp.sum(cmp.astype(f32), axis=1, keepdims=True)

        rank = lax.fori_loop(0, NP // 1024, col_body,
                             jnp.zeros((128, 1), f32))
        nkn_col = nk_col * (rank < kp_col).astype(f32)
        cnt_ref[...] += jnp.dot(_tr(nkn_col), ohc_blk,
                                preferred_element_type=f32)
        nkn_ref[pl.ds(rb, 1), :] = _tr(nkn_col)
        hp_blk = hl_ref[pl.ds(r0, 128), :] * (jnp.tanh(sc_col) * nkn_col)
        hp_ref[pl.ds(r0, 128), :] = hp_blk

        g_lo = jnp.min(bat_row)
        g_hi = jnp.max(bat_row)

        def g_body(g, _):
            m = (bat_col == g) & (nkn_col > 0.0)
            vals = jnp.where(m, hp_blk, jnp.full((128, F), -1e30, f32))
            mrow = jnp.max(vals, axis=0, keepdims=True)
            cur = mx_ref[pl.ds(g, 1), :]
            mx_ref[pl.ds(g, 1), :] = jnp.maximum(cur, mrow)
            return 0

        lax.fori_loop(g_lo, g_hi + 1, g_body, 0)
        return 0

    lax.fori_loop(0, NP // 128, pb_body, 0)

    cnt = _tr(cnt_ref[...])
    seg_sum = jnp.dot(oh, hp_ref[...], preferred_element_type=f32)
    mean = seg_sum / jnp.maximum(cnt, 1.0)
    mx = jnp.where(cnt > 0.0, mx_ref[...], jnp.zeros((G, F), f32))
    x_ref[:, 0:F] = mx
    x_ref[:, F:2 * F] = mean


def _tcc(score_r, batr, batfr, nkr, hl):
    return pl.pallas_call(
        _tcc_body,
        out_shape=[
            jax.ShapeDtypeStruct((NR, 128), f32),
            jax.ShapeDtypeStruct((NP, F), f32),
            jax.ShapeDtypeStruct((G, 2 * F), f32),
        ],
        scratch_shapes=[
            pltpu.VMEM((G, F), f32),
            pltpu.VMEM((1, G), f32),
            pltpu.VMEM((1, G), f32),
        ],
    )(score_r, batr, batfr, nkr, hl)


def _tcd_body(x1_ref, x2_ref, x3_ref, l1_ref, l1b_ref, l2_ref, l2b_ref,
              l3_ref, l3b_ref, o_ref):
    z = x1_ref[...] + x2_ref[...] + x3_ref[...]
    z = jnp.maximum(
        jnp.dot(z, l1_ref[...], preferred_element_type=f32) + l1b_ref[...],
        0.0)
    z = jnp.maximum(
        jnp.dot(z, l2_ref[...], preferred_element_type=f32) + l2b_ref[...],
        0.0)
    z = jnp.dot(z, l3_ref[...], preferred_element_type=f32) + l3b_ref[...]
    m = jnp.max(z, axis=1, keepdims=True)
    lse = m + jnp.log(jnp.sum(jnp.exp(z - m), axis=1, keepdims=True))
    o_ref[...] = z - lse


def _tcd(x1, x2, x3, L1, l1b, L2, l2b, L3, l3b):
    return pl.pallas_call(
        _tcd_body,
        out_shape=jax.ShapeDtypeStruct((G, 2), f32),
    )(x1, x2, x3, L1, l1b.reshape(1, -1), L2, l2b.reshape(1, -1),
      L3, l3b.reshape(1, -1))


def kernel(x, edge_index, edge_attr, batch, W1, b1, Ws1, bs1, W2, b2, Ws2,
           bs2, W3, b3, Ws3, bs3, L1, l1b, L2, l2b, L3, l3b):
    loops = jnp.arange(NP, dtype=jnp.int32)
    npad = EB - E - NP
    src0 = jnp.concatenate([edge_index[0], loops,
                            jnp.zeros((npad,), jnp.int32)])
    dst0 = jnp.concatenate([edge_index[1], loops,
                            jnp.full((npad,), DPAD, jnp.int32)])
    ea0 = jnp.concatenate([edge_attr, jnp.zeros((NP + npad,), f32)])

    srcp, dstp, eap, ek, flagp = _sc_bucket(src0, dst0, ea0)
    src2 = srcp.reshape(EB // 128, 128)
    dst2 = dstp.reshape(EB // 128, 128)

    bat = jnp.concatenate([batch, jnp.full((NP - N,), G - 1, jnp.int32)])
    batr = bat.reshape(1, NP)
    batfr = bat.astype(f32).reshape(1, NP)
    h = jnp.concatenate([x, jnp.zeros((NP - N, F), f32)])
    nk = jnp.ones((NR, 128), f32)

    params = [(W1, b1, Ws1, bs1), (W2, b2, Ws2, bs2), (W3, b3, Ws3, bs3)]
    xs = []
    for (W, b, Ws, bs) in params:
        ek, w, degP, degsP = _sc_edge_deg(srcp, dstp, eap, ek, flagp, nk)
        H, dis, dis_s, invdeg_s = _tca(h, W, degP, degsP)
        (P0,) = _sc_prop128(H, src2, dst2, w.reshape(EB // 128, 128), dis)
        hl, hs, hss = _tcb(P0, b.reshape(1, F), Ws.reshape(1, 1, F), dis_s)
        (S0,) = _sc_prop_scalar(hss, srcp, dstp, ek)
        score = _tcs(S0, hs, dis_s, invdeg_s, bs.reshape(1, 1))
        nk, hp, xl = _tcc(score.reshape(1, NP), batr, batfr,
                          nk.reshape(1, NP), hl)
        h = hp
        xs.append(xl)

    return _tcd(xs[0], xs[1], xs[2], L1, l1b, L2, l2b, L3, l3b)


# speedup vs baseline: 2.2054x; 2.1414x over previous
import functools

import jax
import jax.numpy as jnp
from jax import lax
from jax.experimental import pallas as pl
from jax.experimental.pallas import tpu as pltpu
from jax.experimental.pallas import tpu_sc as plsc

f32 = jnp.float32
i32 = jnp.int32

N = 10000
E = 320000
G = 64
F = 128
NP = 10240
EP = 327680
E2 = 360448
NC = 2
NS = 16
NW = NC * NS
EW = EP // NW
EW2 = E2 // NW
CB = 2048
NR = NP // 128
RPT = NP // NS
JB = 8

_MESH = plsc.VectorSubcoreMesh(
    core_axis_name="c", subcore_axis_name="s", num_cores=NC, num_subcores=NS)
_SC_PARAMS = pltpu.CompilerParams(needs_layout_passes=False)


def _zero16():
    return jnp.zeros((16,), f32)


def _zero_2d(ref, nrows):
    def body(it, _):
        r = it >> 3
        c = (it & 7) * 16
        ref[r, pl.ds(c, 16)] = _zero16()
        return 0
    lax.fori_loop(0, nrows * 8, body, 0)


def _fill_rowidx(ref, nrows):
    def body(it, _):
        ref[pl.ds(it * 16, 16)] = lax.iota(i32, 16) + it * 16
        return 0
    lax.fori_loop(0, nrows // 16, body, 0)


def _rc(v16):
    return lax.shift_right_logical(v16, 7), lax.bitwise_and(v16, 127)


@functools.partial(
    pl.kernel,
    out_type=[
        jax.ShapeDtypeStruct((EP,), f32),
        jax.ShapeDtypeStruct((EP,), f32),
        jax.ShapeDtypeStruct((NC, NR, 128), f32),
        jax.ShapeDtypeStruct((NC, NR, 128), f32),
    ],
    mesh=_MESH,
    compiler_params=_SC_PARAMS,
    scratch_types=[
        pltpu.VMEM((NR, 128), f32),
        pltpu.VMEM((CB,), i32),
        pltpu.VMEM((CB,), i32),
        pltpu.VMEM((CB,), f32),
        pltpu.VMEM((CB,), f32),
        pltpu.VMEM((CB,), f32),
        pltpu.VMEM((CB,), f32),
        pltpu.VMEM((NR, 128), f32),
        pltpu.VMEM((NR, 128), f32),
        pltpu.VMEM((NR, 128), f32),
        pltpu.VMEM((NR,), i32),
        pltpu.VMEM_SHARED((NR, 128), f32),
        pltpu.VMEM_SHARED((NR, 128), f32),
    ],
)
def _sc_edge_deg(src_h, dst_h, ea_h, ekp_h, nk_h,
                 ek_o, w_o, degp_o, degsp_o,
                 nk_v, srcb, dstb, eab, ekpb, ekob, wob,
                 dega, degsa, zbuf, rowidx, degsh, degssh):
    cid = lax.axis_index("c")
    sid = lax.axis_index("s")
    wid = sid * NC + cid

    pltpu.sync_copy(nk_h, nk_v)
    _zero_2d(dega, NR)
    _zero_2d(degsa, NR)
    _fill_rowidx(rowidx, NR)

    @pl.when(sid == 0)
    def _():
        _zero_2d(zbuf, NR)
        pltpu.sync_copy(zbuf, degsh)
        pltpu.sync_copy(zbuf, degssh)

    plsc.subcore_barrier()

    for ci in range(EW // CB):
        base = wid * EW + ci * CB
        pltpu.sync_copy(src_h.at[pl.ds(base, CB)], srcb)
        pltpu.sync_copy(dst_h.at[pl.ds(base, CB)], dstb)
        pltpu.sync_copy(ea_h.at[pl.ds(base, CB)], eab)
        pltpu.sync_copy(ekp_h.at[pl.ds(base, CB)], ekpb)

        def body(j, _):
            o = j * 16
            s16 = srcb[pl.ds(o, 16)]
            d16 = dstb[pl.ds(o, 16)]
            sr, sc = _rc(s16)
            dr, dc = _rc(d16)
            nks = plsc.load_gather(nk_v, [sr, sc])
            nkd = plsc.load_gather(nk_v, [dr, dc])
            ek16 = ekpb[pl.ds(o, 16)] * nks * nkd
            w16 = eab[pl.ds(o, 16)] * ek16
            ekob[pl.ds(o, 16)] = ek16
            wob[pl.ds(o, 16)] = w16
            plsc.addupdate_scatter(dega, [dr, dc], w16)
            plsc.addupdate_scatter(degsa, [dr, dc], ek16)
            return 0

        lax.fori_loop(0, CB // 16, body, 0)
        pltpu.sync_copy(ekob, ek_o.at[pl.ds(base, CB)])
        pltpu.sync_copy(wob, w_o.at[pl.ds(base, CB)])

    pltpu.sync_copy(dega, degsh.at[rowidx], add=True)
    pltpu.sync_copy(degsa, degssh.at[rowidx], add=True)
    plsc.subcore_barrier()

    @pl.when(sid == 0)
    def _():
        pltpu.sync_copy(degsh, degp_o.at[cid])
        pltpu.sync_copy(degssh, degsp_o.at[cid])


@functools.partial(
    pl.kernel,
    out_type=[jax.ShapeDtypeStruct((NC, NP, 128), f32)],
    mesh=_MESH,
    compiler_params=_SC_PARAMS,
    scratch_types=[
        pltpu.VMEM((NR, 128), f32),
        pltpu.VMEM((JB, 128), i32),
        pltpu.VMEM((JB, 128), i32),
        pltpu.VMEM((JB, 128), f32),
        pltpu.VMEM((JB, 128), f32),
        pltpu.VMEM((128, 128), f32),
        pltpu.VMEM((128, 128), f32),
        pltpu.VMEM_SHARED((NP, 128), f32),
        pltpu.SemaphoreType.DMA,
        pltpu.SemaphoreType.DMA,
        pltpu.SemaphoreType.DMA,
        pltpu.SemaphoreType.DMA,
    ],
)
def _sc_prop128(h_h, src2_h, dst2_h, w2_h, dis_h, p0_o,
                dis_v, srcb, dstb, wb, normb, rows0, rows1, acc,
                semg0, semg1, sems0, sems1):
    cid = lax.axis_index("c")
    sid = lax.axis_index("s")
    wid = sid * NC + cid
    rows = (rows0, rows1)
    semg = (semg0, semg1)
    sems = (sems0, sems1)

    pltpu.sync_copy(dis_h, dis_v)
    _zero_2d(rows0, 128)
    for q in range(RPT // 128):
        pltpu.sync_copy(rows0, acc.at[pl.ds(sid * RPT + q * 128, 128)])
    plsc.subcore_barrier()

    for blk in range(EW2 // (JB * 128)):
        rowbase = wid * (EW2 // 128) + blk * JB
        pltpu.sync_copy(src2_h.at[pl.ds(rowbase, JB)], srcb)
        pltpu.sync_copy(dst2_h.at[pl.ds(rowbase, JB)], dstb)
        pltpu.sync_copy(w2_h.at[pl.ds(rowbase, JB)], wb)

        dg0 = pltpu.async_copy(h_h.at[srcb.at[0]], rows[0], semg[0])

        def norm_body(it, _):
            j = it >> 3
            o = (it & 7) * 16
            s16 = srcb[j, pl.ds(o, 16)]
            d16 = dstb[j, pl.ds(o, 16)]
            sr, sc = _rc(s16)
            dr, dc = _rc(d16)
            ds_ = plsc.load_gather(dis_v, [sr, sc])
            dd_ = plsc.load_gather(dis_v, [dr, dc])
            normb[j, pl.ds(o, 16)] = wb[j, pl.ds(o, 16)] * ds_ * dd_
            return 0

        lax.fori_loop(0, JB * 8, norm_body, 0)

        dg = [dg0, None]
        for j in range(JB):
            p = j & 1
            q = (j + 1) & 1
            dg[p].wait()
            if j + 1 < JB:
                dg[q] = pltpu.async_copy(
                    h_h.at[srcb.at[j + 1]], rows[q], semg[q])

            def scale(r, _):
                wspl = plsc.load_gather(
                    normb, [jnp.full((16,), j, i32),
                            jnp.full((16,), 0, i32) + r])
                for c in range(8):
                    rows[p][r, pl.ds(c * 16, 16)] = (
                        rows[p][r, pl.ds(c * 16, 16)] * wspl)
                return 0

            lax.fori_loop(0, 128, scale, 0)
            pltpu.sync_copy(rows[p], acc.at[dstb.at[j]], add=True)

    plsc.subcore_barrier()
    pltpu.sync_copy(acc.at[pl.ds(sid * RPT, RPT)],
                    p0_o.at[cid].at[pl.ds(sid * RPT, RPT)])


@functools.partial(
    pl.kernel,
    out_type=[jax.ShapeDtypeStruct((NC, NR, 128), f32)],
    mesh=_MESH,
    compiler_params=_SC_PARAMS,
    scratch_types=[
        pltpu.VMEM((NR, 128), f32),
        pltpu.VMEM((CB,), i32),
        pltpu.VMEM((CB,), i32),
        pltpu.VMEM((CB,), f32),
        pltpu.VMEM((NR, 128), f32),
        pltpu.VMEM((NR, 128), f32),
        pltpu.VMEM((NR,), i32),
        pltpu.VMEM_SHARED((NR, 128), f32),
    ],
)
def _sc_prop_scalar(hs_h, src_h, dst_h, ek_h, s0_o,
                    hs_v, srcb, dstb, ekb, acc, zbuf, rowidx, accsh):
    cid = lax.axis_index("c")
    sid = lax.axis_index("s")
    wid = sid * NC + cid

    pltpu.sync_copy(hs_h, hs_v)
    _zero_2d(acc, NR)
    _fill_rowidx(rowidx, NR)

    @pl.when(sid == 0)
    def _():
        _zero_2d(zbuf, NR)
        pltpu.sync_copy(zbuf, accsh)

    plsc.subcore_barrier()

    for ci in range(EW // CB):
        base = wid * EW + ci * CB
        pltpu.sync_copy(src_h.at[pl.ds(base, CB)], srcb)
        pltpu.sync_copy(dst_h.at[pl.ds(base, CB)], dstb)
        pltpu.sync_copy(ek_h.at[pl.ds(base, CB)], ekb)

        def body(j, _):
            o = j * 16
            s16 = srcb[pl.ds(o, 16)]
            d16 = dstb[pl.ds(o, 16)]
            sr, sc = _rc(s16)
            dr, dc = _rc(d16)
            v16 = plsc.load_gather(hs_v, [sr, sc]) * ekb[pl.ds(o, 16)]
            plsc.addupdate_scatter(acc, [dr, dc], v16)
            return 0

        lax.fori_loop(0, CB // 16, body, 0)

    pltpu.sync_copy(acc, accsh.at[rowidx], add=True)
    plsc.subcore_barrier()

    @pl.when(sid == 0)
    def _():
        pltpu.sync_copy(accsh, s0_o.at[cid])


def _tca_body(h_ref, w_ref, degp_ref, degsp_ref,
              hh_ref, dis_ref, diss_ref, invds_ref):
    deg = 1.0 + degp_ref[0] + degp_ref[1]
    degs = 1.0 + degsp_ref[0] + degsp_ref[1]
    dis_ref[...] = 1.0 / jnp.sqrt(deg)
    diss = 1.0 / jnp.sqrt(degs)
    diss_ref[...] = diss
    invds_ref[...] = diss * diss
    hh_ref[...] = jnp.dot(h_ref[...], w_ref[...], preferred_element_type=f32)


def _tca(h, W, degp, degsp):
    grid = NP // 1024
    return pl.pallas_call(
        _tca_body,
        grid=(grid,),
        in_specs=[
            pl.BlockSpec((1024, F), lambda i: (i, 0)),
            pl.BlockSpec((F, F), lambda i: (0, 0)),
            pl.BlockSpec((NC, 8, 128), lambda i: (0, i, 0)),
            pl.BlockSpec((NC, 8, 128), lambda i: (0, i, 0)),
        ],
        out_specs=[
            pl.BlockSpec((1024, F), lambda i: (i, 0)),
            pl.BlockSpec((8, 128), lambda i: (i, 0)),
            pl.BlockSpec((8, 128), lambda i: (i, 0)),
            pl.BlockSpec((8, 128), lambda i: (i, 0)),
        ],
        out_shape=[
            jax.ShapeDtypeStruct((NP, F), f32),
            jax.ShapeDtypeStruct((NR, 128), f32),
            jax.ShapeDtypeStruct((NR, 128), f32),
            jax.ShapeDtypeStruct((NR, 128), f32),
        ],
    )(h, W, degp, degsp)


def _tcb_body(p0_ref, b_ref, ws3_ref, diss_ref, hl_ref, hs_ref, hss_ref):
    hl = jnp.maximum(p0_ref[0] + p0_ref[1] + b_ref[...], 0.0)
    hl_ref[...] = hl
    hl3 = hl.reshape(8, 128, F)
    hs = jnp.sum(hl3 * ws3_ref[...], axis=2)
    hs_ref[...] = hs
    hss_ref[...] = hs * diss_ref[...]


def _tcb(P0, b, Ws3, dis_s):
    grid = NP // 1024
    return pl.pallas_call(
        _tcb_body,
        grid=(grid,),
        in_specs=[
            pl.BlockSpec((NC, 1024, F), lambda i: (0, i, 0)),
            pl.BlockSpec((1, F), lambda i: (0, 0)),
            pl.BlockSpec((1, 1, F), lambda i: (0, 0, 0)),
            pl.BlockSpec((8, 128), lambda i: (i, 0)),
        ],
        out_specs=[
            pl.BlockSpec((1024, F), lambda i: (i, 0)),
            pl.BlockSpec((8, 128), lambda i: (i, 0)),
            pl.BlockSpec((8, 128), lambda i: (i, 0)),
        ],
        out_shape=[
            jax.ShapeDtypeStruct((NP, F), f32),
            jax.ShapeDtypeStruct((NR, 128), f32),
            jax.ShapeDtypeStruct((NR, 128), f32),
        ],
    )(P0, b, Ws3, dis_s)


def _tcs_body(s0_ref, hs_ref, diss_ref, invds_ref, bs_ref, sc_ref):
    sc_ref[...] = (diss_ref[...] * (s0_ref[0] + s0_ref[1])
                   + hs_ref[...] * invds_ref[...] + bs_ref[0, 0])


def _tcs(S0, hs, dis_s, invdeg_s, bs2):
    return pl.pallas_call(
        _tcs_body,
        out_shape=jax.ShapeDtypeStruct((NR, 128), f32),
    )(S0, hs, dis_s, invdeg_s, bs2)


def _tr(v):
    return jnp.swapaxes(v, 0, 1)


def _tcc_body(scr_ref, batr_ref, batfr_ref, nkr_ref, hl_ref,
              nkn_ref, hp_ref, x_ref, mx_ref, kc_ref, cnt_ref, gc_ref):
    gids_r = lax.broadcasted_iota(i32, (1, G), 1)
    gids_c = lax.broadcasted_iota(i32, (G, 1), 0).astype(f32)
    oh = (gids_c == batfr_ref[...]).astype(f32)

    mx_ref[...] = jnp.full((G, F), -1e30, f32)
    kc_ref[...] = jnp.zeros((1, G), f32)
    cnt_ref[...] = jnp.zeros((1, G), f32)
    gc_ref[...] = jnp.zeros((1, G), f32)

    ones_row = jnp.full((1, 128), 1.0, f32)

    def pa_body(rb, _):
        r0 = rb * 128
        bat_col = _tr(batr_ref[:, pl.ds(r0, 128)])
        ohc_blk = (bat_col == gids_r).astype(f32)
        nk_row = nkr_ref[:, pl.ds(r0, 128)]
        kc_ref[...] += jnp.dot(nk_row, ohc_blk,
                               preferred_element_type=f32)
        gc_ref[...] += jnp.dot(ones_row, ohc_blk,
                               preferred_element_type=f32)
        return 0

    lax.fori_loop(0, NP // 128, pa_body, 0)
    k_col = _tr(jnp.ceil(0.5 * kc_ref[...]))

    counts_row = gc_ref[...]
    giota_c = lax.broadcasted_iota(i32, (G, 1), 0)
    giota_r2 = lax.broadcasted_iota(i32, (1, G), 1)
    upper = (giota_c < giota_r2).astype(f32)
    starts_row = jnp.dot(counts_row, upper, preferred_element_type=f32)
    ends_row = starts_row + counts_row

    def pb_body(rb, _):
        r0 = rb * 128
        sc_row = scr_ref[:, pl.ds(r0, 128)]
        bat_row = batr_ref[:, pl.ds(r0, 128)]
        nk_row = nkr_ref[:, pl.ds(r0, 128)]
        sc_col = _tr(sc_row)
        bat_col = _tr(bat_row)
        nk_col = _tr(nk_row)
        ohc_blk = (bat_col == gids_r).astype(f32)
        kp_col = jnp.dot(ohc_blk, k_col, preferred_element_type=f32)
        ridx = lax.broadcasted_iota(i32, (128, 1), 0) + r0

        g_lo = jnp.min(bat_row)
        g_hi = jnp.max(bat_row)
        c_lo = jnp.min(jnp.where(gids_r == g_lo, starts_row,
                                 jnp.full((1, G), 1e9, f32))).astype(i32)
        c_hi = jnp.min(jnp.where(gids_r == g_hi, ends_row,
                                 jnp.full((1, G), 1e9, f32))).astype(i32)

        def col_body(cb, acc):
            c0 = cb * 1024
            sc_c = scr_ref[:, pl.ds(c0, 1024)]
            bat_c = batr_ref[:, pl.ds(c0, 1024)]
            keep_c = nkr_ref[:, pl.ds(c0, 1024)]
            cidx = lax.broadcasted_iota(i32, (1, 1024), 1) + c0
            before = (sc_c > sc_col) | ((sc_c == sc_col) & (cidx < ridx))
            cmp = ((bat_c == bat_col) & (keep_c > 0.0) & before)
            return acc + jnp.sum(cmp.astype(f32), axis=1, keepdims=True)

        rank = lax.fori_loop(lax.shift_right_logical(c_lo, 10),
                             lax.shift_right_logical(c_hi + 1023, 10),
                             col_body, jnp.zeros((128, 1), f32))
        nkn_col = nk_col * (rank < kp_col).astype(f32)
        cnt_ref[...] += jnp.dot(_tr(nkn_col), ohc_blk,
                                preferred_element_type=f32)
        nkn_ref[pl.ds(rb, 1), :] = _tr(nkn_col)
        hp_blk = hl_ref[pl.ds(r0, 128), :] * (jnp.tanh(sc_col) * nkn_col)
        hp_ref[pl.ds(r0, 128), :] = hp_blk

        def g_body(g, _):
            m = (bat_col == g) & (nkn_col > 0.0)
            vals = jnp.where(m, hp_blk, jnp.full((128, F), -1e30, f32))
            mrow = jnp.max(vals, axis=0, keepdims=True)
            cur = mx_ref[pl.ds(g, 1), :]
            mx_ref[pl.ds(g, 1), :] = jnp.maximum(cur, mrow)
            return 0

        lax.fori_loop(g_lo, g_hi + 1, g_body, 0)
        return 0

    lax.fori_loop(0, NP // 128, pb_body, 0)

    cnt = _tr(cnt_ref[...])
    seg_sum = jnp.dot(oh, hp_ref[...], preferred_element_type=f32)
    mean = seg_sum / jnp.maximum(cnt, 1.0)
    mx = jnp.where(cnt > 0.0, mx_ref[...], jnp.zeros((G, F), f32))
    x_ref[:, 0:F] = mx
    x_ref[:, F:2 * F] = mean


def _tcc(score_r, batr, batfr, nkr, hl):
    return pl.pallas_call(
        _tcc_body,
        out_shape=[
            jax.ShapeDtypeStruct((NR, 128), f32),
            jax.ShapeDtypeStruct((NP, F), f32),
            jax.ShapeDtypeStruct((G, 2 * F), f32),
        ],
        scratch_shapes=[
            pltpu.VMEM((G, F), f32),
            pltpu.VMEM((1, G), f32),
            pltpu.VMEM((1, G), f32),
            pltpu.VMEM((1, G), f32),
        ],
    )(score_r, batr, batfr, nkr, hl)


def _tcd_body(x1_ref, x2_ref, x3_ref, l1_ref, l1b_ref, l2_ref, l2b_ref,
              l3_ref, l3b_ref, o_ref):
    z = x1_ref[...] + x2_ref[...] + x3_ref[...]
    z = jnp.maximum(
        jnp.dot(z, l1_ref[...], preferred_element_type=f32) + l1b_ref[...],
        0.0)
    z = jnp.maximum(
        jnp.dot(z, l2_ref[...], preferred_element_type=f32) + l2b_ref[...],
        0.0)
    z = jnp.dot(z, l3_ref[...], preferred_element_type=f32) + l3b_ref[...]
    m = jnp.max(z, axis=1, keepdims=True)
    lse = m + jnp.log(jnp.sum(jnp.exp(z - m), axis=1, keepdims=True))
    o_ref[...] = z - lse


def _tcd(x1, x2, x3, L1, l1b, L2, l2b, L3, l3b):
    return pl.pallas_call(
        _tcd_body,
        out_shape=jax.ShapeDtypeStruct((G, 2), f32),
    )(x1, x2, x3, L1, l1b.reshape(1, -1), L2, l2b.reshape(1, -1),
      L3, l3b.reshape(1, -1))


def kernel(x, edge_index, edge_attr, batch, W1, b1, Ws1, bs1, W2, b2, Ws2,
           bs2, W3, b3, Ws3, bs3, L1, l1b, L2, l2b, L3, l3b):
    pad_e = jnp.full((EP - E,), NP - 1, jnp.int32)
    src = jnp.concatenate([edge_index[0], pad_e])
    dst = jnp.concatenate([edge_index[1], pad_e])
    ea = jnp.concatenate([edge_attr, jnp.zeros((EP - E,), f32)])
    loops = jnp.arange(NP, dtype=jnp.int32)
    pad2 = jnp.full((E2 - E - NP,), NP - 1, jnp.int32)
    src2 = jnp.concatenate([edge_index[0], loops, pad2]).reshape(E2 // 128, 128)
    dst2 = jnp.concatenate([edge_index[1], loops, pad2]).reshape(E2 // 128, 128)
    w2_tail = jnp.concatenate(
        [jnp.ones((NP,), f32), jnp.zeros((E2 - E - NP,), f32)])

    bat = jnp.concatenate([batch, jnp.full((NP - N,), G - 1, jnp.int32)])
    batr = bat.reshape(1, NP)
    batfr = bat.astype(f32).reshape(1, NP)
    h = jnp.concatenate([x, jnp.zeros((NP - N, F), f32)])
    ek = jnp.concatenate([jnp.ones((E,), f32), jnp.zeros((EP - E,), f32)])
    nk = jnp.ones((NR, 128), f32)

    params = [(W1, b1, Ws1, bs1), (W2, b2, Ws2, bs2), (W3, b3, Ws3, bs3)]
    xs = []
    for (W, b, Ws, bs) in params:
        ek, w, degP, degsP = _sc_edge_deg(src, dst, ea, ek, nk)
        H, dis, dis_s, invdeg_s = _tca(h, W, degP, degsP)
        w2 = jnp.concatenate([w[:E], w2_tail]).reshape(E2 // 128, 128)
        (P0,) = _sc_prop128(H, src2, dst2, w2, dis)
        hl, hs, hss = _tcb(P0, b.reshape(1, F), Ws.reshape(1, 1, F), dis_s)
        (S0,) = _sc_prop_scalar(hss, src, dst, ek)
        score = _tcs(S0, hs, dis_s, invdeg_s, bs.reshape(1, 1))
        nk, hp, xl = _tcc(score.reshape(1, NP), batr, batfr,
                          nk.reshape(1, NP), hl)
        h = hp
        xs.append(xl)

    return _tcd(xs[0], xs[1], xs[2], L1, l1b, L2, l2b, L3, l3b)
